# Initial kernel scaffold; baseline (speedup 1.0000x reference)
import jax
import jax.numpy as jnp
from jax.experimental import pallas as pl


def kernel(user_emb, item_emb, adj_values, adj_indices):
    raise NotImplementedError("write your pallas kernel here")


# speedup vs baseline: 3.3238x; 3.3238x over previous
import functools

import numpy as np

import jax
import jax.numpy as jnp
from jax import lax
from jax.experimental import pallas as pl
from jax.experimental.pallas import tpu as pltpu
from jax.experimental.pallas import tpu_sc as plsc

USER_NUM = 25000
ITEM_NUM = 25000
N_NODES = USER_NUM + ITEM_NUM
N_EDGES = 800000
EMB = 64
HALF = 32
N_LAYERS = 3

NC = 2
NS = 16
EBLK = 128
NBLK = 391
E_PAD = NS * NBLK * EBLK
EDGES_PER_TILE = NBLK * EBLK
N_PAD = 50048
ROWS_PER_TILE = N_PAD // NS
ZROWS = 184
ZCOPIES = ROWS_PER_TILE // ZROWS
UROWS = 1568
UCHUNK = 112
UCHUNKS = UROWS // UCHUNK

def _body(ego0, src, dst, val, out, uout, acc, zb, src_v, dst_v, val_v,
          rows_v, mba, mbb, sem):
    c = lax.axis_index("c")
    s = lax.axis_index("s")

    z16 = jnp.zeros((16,), jnp.float32)

    def zb_init(i, _):
        zb[i, pl.ds(0, 16)] = z16
        zb[i, pl.ds(16, 16)] = z16
        return 0
    lax.fori_loop(0, ZROWS, zb_init, 0)

    def zero_acc():
        for k in range(ZCOPIES):
            pltpu.sync_copy(zb, acc.at[pl.ds(s * ROWS_PER_TILE + k * ZROWS,
                                             ZROWS)])

    zero_acc()
    plsc.subcore_barrier()

    for l in range(N_LAYERS):
        table = ego0.at[c] if l == 0 else out.at[l - 1, c]

        def edge_block(i, _):
            off = s * EDGES_PER_TILE + i * EBLK
            pltpu.sync_copy(src.at[pl.ds(off, EBLK)], src_v)
            pltpu.sync_copy(dst.at[pl.ds(off, EBLK)], dst_v)
            pltpu.sync_copy(val.at[pl.ds(off, EBLK)], val_v)
            pltpu.async_copy(table.at[src_v], rows_v, sem).wait()
            for k in range(EBLK // 16):
                vv = val_v[pl.ds(k * 16, 16)]
                for j in range(16):
                    e = k * 16 + j
                    sv = jnp.broadcast_to(vv[j], (16,))
                    rows_v[e, pl.ds(0, 16)] = rows_v[e, pl.ds(0, 16)] * sv
                    rows_v[e, pl.ds(16, 16)] = rows_v[e, pl.ds(16, 16)] * sv
            pltpu.sync_copy(rows_v, acc.at[dst_v], add=True)
            return 0

        lax.fori_loop(0, NBLK, edge_block, 0)
        plsc.subcore_barrier()

        pltpu.sync_copy(acc.at[pl.ds(s * ROWS_PER_TILE, ROWS_PER_TILE)],
                        out.at[l, c].at[pl.ds(s * ROWS_PER_TILE,
                                              ROWS_PER_TILE)])
        if l < N_LAYERS - 1:
            zero_acc()
        plsc.subcore_barrier()

    def mean_rows(r0, nrows, ba, bb):
        pltpu.sync_copy(ego0.at[c].at[pl.ds(r0, nrows)], ba)

        for l in range(N_LAYERS):
            pltpu.sync_copy(out.at[l, c].at[pl.ds(r0, nrows)], bb)
            scale = 0.25 if l == N_LAYERS - 1 else 1.0

            def add_row(i, _):
                for j in range(2):
                    a = ba[i, pl.ds(j * 16, 16)]
                    b = bb[i, pl.ds(j * 16, 16)]
                    ba[i, pl.ds(j * 16, 16)] = (a + b) * scale
                return 0
            lax.fori_loop(0, nrows, add_row, 0)
        pltpu.sync_copy(ba, uout.at[c].at[pl.ds(r0, nrows)])

    def mean_chunk(i, _):
        mean_rows(s * UROWS + i * UCHUNK, UCHUNK, mba, mbb)
        return 0
    lax.fori_loop(0, UCHUNKS, mean_chunk, 0)


@jax.jit
def _propagate(ego0, src, dst, val):
    f = pl.kernel(
        _body,
        out_type=(
            jax.ShapeDtypeStruct((N_LAYERS, NC, N_PAD, HALF), jnp.float32),
            jax.ShapeDtypeStruct((NC, UROWS * NS, HALF), jnp.float32),
        ),
        mesh=plsc.VectorSubcoreMesh(core_axis_name="c", subcore_axis_name="s",
                                    num_cores=NC, num_subcores=NS),
        compiler_params=pltpu.CompilerParams(use_tc_tiling_on_sc=False),
        scratch_types=[
            pltpu.VMEM_SHARED((N_PAD, HALF), jnp.float32),
            pltpu.VMEM((ZROWS, HALF), jnp.float32),
            pltpu.VMEM((EBLK,), jnp.int32),
            pltpu.VMEM((EBLK,), jnp.int32),
            pltpu.VMEM((EBLK,), jnp.float32),
            pltpu.VMEM((EBLK, HALF), jnp.float32),
            pltpu.VMEM((UCHUNK, HALF), jnp.float32),
            pltpu.VMEM((UCHUNK, HALF), jnp.float32),
            pltpu.SemaphoreType.DMA,
        ],
    )
    return f(ego0, src, dst, val)


def kernel(user_emb, item_emb, adj_values, adj_indices):
    zpad = jnp.zeros((N_PAD - N_NODES, HALF), jnp.float32)
    ego0 = jnp.stack([
        jnp.concatenate([user_emb[:, :HALF], item_emb[:, :HALF], zpad], axis=0),
        jnp.concatenate([user_emb[:, HALF:], item_emb[:, HALF:], zpad], axis=0),
    ])

    dst = adj_indices[0].astype(jnp.int32)
    src = adj_indices[1].astype(jnp.int32)
    val = adj_values.astype(jnp.float32)

    npad = E_PAD - N_EDGES
    spread = (jnp.arange(npad, dtype=jnp.int32) * 63) % N_NODES
    src = jnp.concatenate([src, spread])
    dst = jnp.concatenate([dst, spread])
    val = jnp.concatenate([val, jnp.zeros((npad,), jnp.float32)])

    out, uout = _propagate(ego0, src, dst, val)

    user_out = jnp.concatenate([uout[0, :USER_NUM], uout[1, :USER_NUM]],
                               axis=-1)
    item_layers = [item_emb] + [
        jnp.concatenate([out[l, 0, USER_NUM:N_NODES],
                         out[l, 1, USER_NUM:N_NODES]], axis=-1)
        for l in range(N_LAYERS)
    ]
    item_out = jnp.stack(item_layers, axis=0)
    return (user_out, item_out)


# speedup vs baseline: 6.8371x; 2.0570x over previous
import functools

import jax
import jax.numpy as jnp
from jax import lax
from jax.experimental import pallas as pl
from jax.experimental.pallas import tpu as pltpu
from jax.experimental.pallas import tpu_sc as plsc

USER_NUM = 25000
ITEM_NUM = 25000
N_NODES = USER_NUM + ITEM_NUM
N_EDGES = 800000
EMB = 64
HALF = 32
N_LAYERS = 3

NC = 2
NS = 16
EBLK = 128
NBLK = 393
NBLK_IO = NBLK + 1
E_PAD = NS * NBLK * EBLK
N_PAD = 50048
ROWS_PER_TILE = N_PAD // NS
ZROWS = 184
ZCOPIES = ROWS_PER_TILE // ZROWS
UROWS = 1568
UCHUNK = 112
UCHUNKS = UROWS // UCHUNK


def _body(ego0, srcs, dv, out, uout, acc, sb0, sb1, sb2, db0, db1, db2,
          dstb0, dstb1, dstb2, rows, zb, mba, mbb,
          si0, si1, si2, sj0, sj1, sj2, sg0, sg1, sg2, ss0, ss1, ss2):
    c = lax.axis_index("c")
    s = lax.axis_index("s")
    sbuf = (sb0, sb1, sb2)
    dvb = (db0, db1, db2)
    dstb = (dstb0, dstb1, dstb2)
    sem_i = (si0, si1, si2)
    sem_j = (sj0, sj1, sj2)
    sem_g = (sg0, sg1, sg2)
    sem_s = (ss0, ss1, ss2)

    z16 = jnp.zeros((16,), jnp.float32)

    def zb_init(i, _):
        zb[i, pl.ds(0, 16)] = z16
        zb[i, pl.ds(16, 16)] = z16
        return 0
    lax.fori_loop(0, ZROWS, zb_init, 0)

    def zero_acc():
        for k in range(ZCOPIES):
            pltpu.sync_copy(zb, acc.at[pl.ds(s * ROWS_PER_TILE + k * ZROWS,
                                             ZROWS)])

    zero_acc()
    plsc.subcore_barrier()

    class idx_copy:
        def __init__(self, j, m):
            g = s * NBLK_IO + j
            self.a = pltpu.make_async_copy(srcs.at[g], sbuf[m], sem_i[m])
            self.b = pltpu.make_async_copy(dv.at[g], dvb[m], sem_j[m])

        def start(self):
            self.a.start()
            self.b.start()

        def wait(self):
            self.a.wait()
            self.b.wait()

    def gat_copy(table, m):
        return pltpu.make_async_copy(table.at[sbuf[m]], rows.at[m],
                                     sem_g[m])

    def sct_copy(m):
        return pltpu.make_async_copy(rows.at[m], acc.at[dstb[m]], sem_s[m])

    def scale(m):
        for k in range(EBLK // 16):
            sl = pl.ds(k * 16, 16)
            dstb[m][sl] = dvb[m][0, sl]
            vv = dvb[m].bitcast(jnp.float32)[1, sl]
            for q in range(16):
                e = k * 16 + q
                sv = jnp.broadcast_to(vv[q], (16,))
                rows[m, e, pl.ds(0, 16)] = rows[m, e, pl.ds(0, 16)] * sv
                rows[m, e, pl.ds(16, 16)] = rows[m, e, pl.ds(16, 16)] * sv

    def edge_phase(table):
        idx_copy(0, 0).start()
        idx_copy(1, 1).start()
        idx_copy(0, 0).wait()
        gat_copy(table, 0).start()

        gat_copy(table, 0).wait()
        scale(0)
        sct_copy(0).start(add=True)
        idx_copy(2, 2).start()
        idx_copy(1, 1).wait()
        gat_copy(table, 1).start()

        gat_copy(table, 1).wait()
        scale(1)
        sct_copy(1).start(add=True)
        idx_copy(3, 0).start()
        idx_copy(2, 2).wait()
        gat_copy(table, 2).start()

        def outer(t, _):
            jb = 2 + 3 * t
            for u in range(3):
                j = jb + u
                m = (2 + u) % 3
                m1 = u % 3
                m2 = (1 + u) % 3
                gat_copy(table, m).wait()
                scale(m)
                sct_copy(m).start(add=True)
                sct_copy(m1).wait()
                idx_copy(j + 2, m2).start()
                idx_copy(j + 1, m1).wait()
                gat_copy(table, m1).start()
            return 0
        lax.fori_loop(0, (NBLK - 3) // 3, outer, 0)

        gat_copy(table, 2).wait()
        scale(2)
        sct_copy(2).start(add=True)
        idx_copy(NBLK, 0).wait()
        sct_copy(0).wait()
        sct_copy(1).wait()
        sct_copy(2).wait()

    for l in range(N_LAYERS):
        table = ego0.at[c] if l == 0 else out.at[l - 1, c]
        edge_phase(table)
        plsc.subcore_barrier()

        pltpu.sync_copy(acc.at[pl.ds(s * ROWS_PER_TILE, ROWS_PER_TILE)],
                        out.at[l, c].at[pl.ds(s * ROWS_PER_TILE,
                                              ROWS_PER_TILE)])
        if l < N_LAYERS - 1:
            zero_acc()
        plsc.subcore_barrier()

    def mean_rows(r0, nrows, ba, bb):
        pltpu.sync_copy(ego0.at[c].at[pl.ds(r0, nrows)], ba)

        for l in range(N_LAYERS):
            pltpu.sync_copy(out.at[l, c].at[pl.ds(r0, nrows)], bb)
            scalef = 0.25 if l == N_LAYERS - 1 else 1.0

            def add_row(i, _):
                for j in range(2):
                    a = ba[i, pl.ds(j * 16, 16)]
                    b = bb[i, pl.ds(j * 16, 16)]
                    ba[i, pl.ds(j * 16, 16)] = (a + b) * scalef
                return 0
            lax.fori_loop(0, nrows, add_row, 0)
        pltpu.sync_copy(ba, uout.at[c].at[pl.ds(r0, nrows)])

    def mean_chunk(i, _):
        mean_rows(s * UROWS + i * UCHUNK, UCHUNK, mba, mbb)
        return 0
    lax.fori_loop(0, UCHUNKS, mean_chunk, 0)


@jax.jit
def _propagate(ego0, srcs, dv):
    f = pl.kernel(
        _body,
        out_type=(
            jax.ShapeDtypeStruct((N_LAYERS, NC, N_PAD, HALF), jnp.float32),
            jax.ShapeDtypeStruct((NC, UROWS * NS, HALF), jnp.float32),
        ),
        mesh=plsc.VectorSubcoreMesh(core_axis_name="c", subcore_axis_name="s",
                                    num_cores=NC, num_subcores=NS),
        compiler_params=pltpu.CompilerParams(use_tc_tiling_on_sc=False),
        scratch_types=[
            pltpu.VMEM_SHARED((N_PAD, HALF), jnp.float32),
            pltpu.VMEM((EBLK,), jnp.int32),
            pltpu.VMEM((EBLK,), jnp.int32),
            pltpu.VMEM((EBLK,), jnp.int32),
            pltpu.VMEM((2, EBLK), jnp.int32),
            pltpu.VMEM((2, EBLK), jnp.int32),
            pltpu.VMEM((2, EBLK), jnp.int32),
            pltpu.VMEM((EBLK,), jnp.int32),
            pltpu.VMEM((EBLK,), jnp.int32),
            pltpu.VMEM((EBLK,), jnp.int32),
            pltpu.VMEM((3, EBLK, HALF), jnp.float32),
            pltpu.VMEM((ZROWS, HALF), jnp.float32),
            pltpu.VMEM((UCHUNK, HALF), jnp.float32),
            pltpu.VMEM((UCHUNK, HALF), jnp.float32),
        ] + [pltpu.SemaphoreType.DMA] * 12,
    )
    return f(ego0, srcs, dv)


def kernel(user_emb, item_emb, adj_values, adj_indices):
    zpad = jnp.zeros((N_PAD - N_NODES, HALF), jnp.float32)
    ego0 = jnp.stack([
        jnp.concatenate([user_emb[:, :HALF], item_emb[:, :HALF], zpad], axis=0),
        jnp.concatenate([user_emb[:, HALF:], item_emb[:, HALF:], zpad], axis=0),
    ])

    dst = adj_indices[0].astype(jnp.int32)
    src = adj_indices[1].astype(jnp.int32)
    val = adj_values.astype(jnp.float32)

    npad = E_PAD - N_EDGES
    spread = (jnp.arange(npad, dtype=jnp.int32) * 63) % N_NODES
    src = jnp.concatenate([src, spread])
    dst = jnp.concatenate([dst, spread])
    vbits = lax.bitcast_convert_type(
        jnp.concatenate([val, jnp.zeros((npad,), jnp.float32)]), jnp.int32)

    sb = src.reshape(NS, NBLK, EBLK)
    sb = jnp.pad(sb, ((0, 0), (0, 1), (0, 0))).reshape(NS * NBLK_IO, EBLK)
    db = jnp.stack([dst, vbits])
    db = db.reshape(2, NS, NBLK, EBLK).transpose(1, 2, 0, 3)
    db = jnp.pad(db, ((0, 0), (0, 1), (0, 0), (0, 0)))
    db = db.reshape(NS * NBLK_IO, 2, EBLK)

    out, uout = _propagate(ego0, sb, db)

    user_out = jnp.concatenate([uout[0, :USER_NUM], uout[1, :USER_NUM]],
                               axis=-1)
    item_layers = [item_emb] + [
        jnp.concatenate([out[l, 0, USER_NUM:N_NODES],
                         out[l, 1, USER_NUM:N_NODES]], axis=-1)
        for l in range(N_LAYERS)
    ]
    item_out = jnp.stack(item_layers, axis=0)
    return (user_out, item_out)


# speedup vs baseline: 9.2869x; 1.3583x over previous
import functools

import jax
import jax.numpy as jnp
from jax import lax
from jax.experimental import pallas as pl
from jax.experimental.pallas import tpu as pltpu
from jax.experimental.pallas import tpu_sc as plsc

USER_NUM = 25000
ITEM_NUM = 25000
N_NODES = USER_NUM + ITEM_NUM
N_EDGES = 800000
EMB = 64
HALF = 32
N_LAYERS = 3

NC = 2
NS = 16
EBLK = 128
NBLK = 393
NBLK_IO = NBLK + 1
E_PAD = NS * NBLK * EBLK
N_PAD = 50048
ROWS_PER_TILE = N_PAD // NS
ZROWS = 184
ZCOPIES = ROWS_PER_TILE // ZROWS
UROWS = 1568
UCHUNK = 112
UCHUNKS = UROWS // UCHUNK


def _body(ego0, srcs, dv, out, uout, acc, sb0, sb1, sb2, db0, db1, db2,
          dstb0, dstb1, dstb2, rows, zb, mba, mbb,
          si0, si1, si2, sj0, sj1, sj2, sg0, sg1, sg2, ss0, ss1, ss2):
    c = lax.axis_index("c")
    s = lax.axis_index("s")
    sbuf = (sb0, sb1, sb2)
    dvb = (db0, db1, db2)
    dstb = (dstb0, dstb1, dstb2)
    sem_i = (si0, si1, si2)
    sem_j = (sj0, sj1, sj2)
    sem_g = (sg0, sg1, sg2)
    sem_s = (ss0, ss1, ss2)

    z16 = jnp.zeros((16,), jnp.float32)

    def zb_init(i, _):
        zb[i, pl.ds(0, 16)] = z16
        zb[i, pl.ds(16, 16)] = z16
        return 0
    lax.fori_loop(0, ZROWS, zb_init, 0)

    def zero_acc():
        for k in range(ZCOPIES):
            pltpu.sync_copy(zb, acc.at[pl.ds(s * ROWS_PER_TILE + k * ZROWS,
                                             ZROWS)])

    zero_acc()
    plsc.subcore_barrier()

    class idx_copy:
        def __init__(self, j, m):
            g = s * NBLK_IO + j
            self.a = pltpu.make_async_copy(srcs.at[g], sbuf[m], sem_i[m])
            self.b = pltpu.make_async_copy(dv.at[g], dvb[m], sem_j[m])

        def start(self):
            self.a.start()
            self.b.start()

        def wait(self):
            self.a.wait()
            self.b.wait()

    def gat_copy(table, m):
        return pltpu.make_async_copy(table.at[sbuf[m]], rows.at[m],
                                     sem_g[m])

    def sct_copy(m):
        return pltpu.make_async_copy(rows.at[m], acc.at[dstb[m]], sem_s[m])

    def scale(m):
        for k in range(EBLK // 16):
            sl = pl.ds(k * 16, 16)
            dstb[m][sl] = dvb[m][0, sl]
            vv = dvb[m].bitcast(jnp.float32)[1, sl]
            for q in range(16):
                e = k * 16 + q
                sv = jnp.broadcast_to(vv[q], (16,))
                rows[m, e, pl.ds(0, 16)] = rows[m, e, pl.ds(0, 16)] * sv
                rows[m, e, pl.ds(16, 16)] = rows[m, e, pl.ds(16, 16)] * sv

    def edge_phase(table):
        idx_copy(0, 0).start()
        idx_copy(1, 1).start()
        idx_copy(0, 0).wait()
        gat_copy(table, 0).start()

        idx_copy(1, 1).wait()
        gat_copy(table, 1).start()
        idx_copy(2, 2).start()
        gat_copy(table, 0).wait()
        scale(0)
        sct_copy(0).start(add=True)

        idx_copy(2, 2).wait()
        gat_copy(table, 2).start()
        idx_copy(3, 0).start()
        gat_copy(table, 1).wait()
        scale(1)
        sct_copy(1).start(add=True)

        def outer(t, _):
            jb = 2 + 3 * t
            for u in range(3):
                j = jb + u
                m = (2 + u) % 3
                m1 = u % 3
                m2 = (1 + u) % 3
                sct_copy(m1).wait()
                idx_copy(j + 1, m1).wait()
                gat_copy(table, m1).start()
                idx_copy(j + 2, m2).start()
                gat_copy(table, m).wait()
                scale(m)
                sct_copy(m).start(add=True)
            return 0
        lax.fori_loop(0, (NBLK - 3) // 3, outer, 0)

        sct_copy(0).wait()
        idx_copy(NBLK, 0).wait()
        gat_copy(table, 2).wait()
        scale(2)
        sct_copy(2).start(add=True)
        sct_copy(1).wait()
        sct_copy(2).wait()

    for l in range(N_LAYERS):
        table = ego0.at[c] if l == 0 else out.at[l - 1, c]
        edge_phase(table)
        plsc.subcore_barrier()

        pltpu.sync_copy(acc.at[pl.ds(s * ROWS_PER_TILE, ROWS_PER_TILE)],
                        out.at[l, c].at[pl.ds(s * ROWS_PER_TILE,
                                              ROWS_PER_TILE)])
        if l < N_LAYERS - 1:
            zero_acc()
        plsc.subcore_barrier()

    def mean_rows(r0, nrows, ba, bb):
        pltpu.sync_copy(ego0.at[c].at[pl.ds(r0, nrows)], ba)

        for l in range(N_LAYERS):
            pltpu.sync_copy(out.at[l, c].at[pl.ds(r0, nrows)], bb)
            scalef = 0.25 if l == N_LAYERS - 1 else 1.0

            def add_row(i, _):
                for j in range(2):
                    a = ba[i, pl.ds(j * 16, 16)]
                    b = bb[i, pl.ds(j * 16, 16)]
                    ba[i, pl.ds(j * 16, 16)] = (a + b) * scalef
                return 0
            lax.fori_loop(0, nrows, add_row, 0)
        pltpu.sync_copy(ba, uout.at[c].at[pl.ds(r0, nrows)])

    def mean_chunk(i, _):
        mean_rows(s * UROWS + i * UCHUNK, UCHUNK, mba, mbb)
        return 0
    lax.fori_loop(0, UCHUNKS, mean_chunk, 0)


@jax.jit
def _propagate(ego0, srcs, dv):
    f = pl.kernel(
        _body,
        out_type=(
            jax.ShapeDtypeStruct((N_LAYERS, NC, N_PAD, HALF), jnp.float32),
            jax.ShapeDtypeStruct((NC, UROWS * NS, HALF), jnp.float32),
        ),
        mesh=plsc.VectorSubcoreMesh(core_axis_name="c", subcore_axis_name="s",
                                    num_cores=NC, num_subcores=NS),
        compiler_params=pltpu.CompilerParams(use_tc_tiling_on_sc=False),
        scratch_types=[
            pltpu.VMEM_SHARED((N_PAD, HALF), jnp.float32),
            pltpu.VMEM((EBLK,), jnp.int32),
            pltpu.VMEM((EBLK,), jnp.int32),
            pltpu.VMEM((EBLK,), jnp.int32),
            pltpu.VMEM((2, EBLK), jnp.int32),
            pltpu.VMEM((2, EBLK), jnp.int32),
            pltpu.VMEM((2, EBLK), jnp.int32),
            pltpu.VMEM((EBLK,), jnp.int32),
            pltpu.VMEM((EBLK,), jnp.int32),
            pltpu.VMEM((EBLK,), jnp.int32),
            pltpu.VMEM((3, EBLK, HALF), jnp.float32),
            pltpu.VMEM((ZROWS, HALF), jnp.float32),
            pltpu.VMEM((UCHUNK, HALF), jnp.float32),
            pltpu.VMEM((UCHUNK, HALF), jnp.float32),
        ] + [pltpu.SemaphoreType.DMA] * 12,
    )
    return f(ego0, srcs, dv)


def kernel(user_emb, item_emb, adj_values, adj_indices):
    zpad = jnp.zeros((N_PAD - N_NODES, HALF), jnp.float32)
    ego0 = jnp.stack([
        jnp.concatenate([user_emb[:, :HALF], item_emb[:, :HALF], zpad], axis=0),
        jnp.concatenate([user_emb[:, HALF:], item_emb[:, HALF:], zpad], axis=0),
    ])

    dst = adj_indices[0].astype(jnp.int32)
    src = adj_indices[1].astype(jnp.int32)
    val = adj_values.astype(jnp.float32)

    npad = E_PAD - N_EDGES
    spread = (jnp.arange(npad, dtype=jnp.int32) * 63) % N_NODES
    src = jnp.concatenate([src, spread])
    dst = jnp.concatenate([dst, spread])
    vbits = lax.bitcast_convert_type(
        jnp.concatenate([val, jnp.zeros((npad,), jnp.float32)]), jnp.int32)

    sb = src.reshape(NS, NBLK, EBLK)
    sb = jnp.pad(sb, ((0, 0), (0, 1), (0, 0))).reshape(NS * NBLK_IO, EBLK)
    db = jnp.stack([dst, vbits])
    db = db.reshape(2, NS, NBLK, EBLK).transpose(1, 2, 0, 3)
    db = jnp.pad(db, ((0, 0), (0, 1), (0, 0), (0, 0)))
    db = db.reshape(NS * NBLK_IO, 2, EBLK)

    out, uout = _propagate(ego0, sb, db)

    user_out = jnp.concatenate([uout[0, :USER_NUM], uout[1, :USER_NUM]],
                               axis=-1)
    item_layers = [item_emb] + [
        jnp.concatenate([out[l, 0, USER_NUM:N_NODES],
                         out[l, 1, USER_NUM:N_NODES]], axis=-1)
        for l in range(N_LAYERS)
    ]
    item_out = jnp.stack(item_layers, axis=0)
    return (user_out, item_out)
